# Initial kernel scaffold; baseline (speedup 1.0000x reference)
import jax
import jax.numpy as jnp
from jax.experimental import pallas as pl


def kernel(tile_type, unit_counts_player_0, relic_map, normalized_reward_last_round, embed_table):
    raise NotImplementedError("write your pallas kernel here")


# speedup vs baseline: 4.5648x; 4.5648x over previous
import functools

import jax
import jax.numpy as jnp
from jax import lax
from jax.experimental import pallas as pl
from jax.experimental.pallas import tpu as pltpu
from jax.experimental.pallas import tpu_sc as plsc

B, P, H, W = 4096, 2, 24, 24
C = 7
E = B * P
S = H * W
NW = 32
EPW = E // NW
CE = 8
NCH = EPW // CE
GROUPS = S // 16
L = 16


def _body(tt_hbm, uc_hbm, rm_hbm, rw_hbm, tab_hbm, out_hbm,
          tt_v, uc_v, rm_v, rw_v, tab_v, out_v):
    wid = lax.axis_index("s") * 2 + lax.axis_index("c")
    pltpu.sync_copy(tab_hbm, tab_v)
    pltpu.sync_copy(rw_hbm.at[pl.ds(wid * EPW * L, EPW * L)], rw_v)
    iota7 = lax.iota(jnp.int32, L) * 7

    def chunk_body(ch, carry):
        base_env = wid * EPW + ch * CE
        pltpu.sync_copy(tt_hbm.at[pl.ds(base_env * S, CE * S)], tt_v)
        pltpu.sync_copy(uc_hbm.at[pl.ds(base_env * S, CE * S)], uc_v)
        pltpu.sync_copy(rm_hbm.at[pl.ds(base_env * S, CE * S)], rm_v)

        def env_body(e, carry2):
            rv = rw_v[pl.ds((ch * CE + e) * L, L)]
            for g in range(GROUPS):
                o = e * S + g * L
                tt = tt_v[pl.ds(o, L)]
                uc = uc_v[pl.ds(o, L)]
                rm = rm_v[pl.ds(o, L)]
                tt4 = tt * 4
                obase = iota7 + (o * C)
                e0 = plsc.load_gather(tab_v, [tt4])
                e1 = plsc.load_gather(tab_v, [tt4 + 1])
                e2 = plsc.load_gather(tab_v, [tt4 + 2])
                e3 = plsc.load_gather(tab_v, [tt4 + 3])
                plsc.store_scatter(out_v, [obase], e0)
                plsc.store_scatter(out_v, [obase + 1], e1)
                plsc.store_scatter(out_v, [obase + 2], e2)
                plsc.store_scatter(out_v, [obase + 3], e3)
                plsc.store_scatter(out_v, [obase + 4], uc)
                plsc.store_scatter(out_v, [obase + 5], rm.astype(jnp.float32))
                plsc.store_scatter(out_v, [obase + 6], rv)
            return carry2

        lax.fori_loop(0, CE, env_body, 0)
        pltpu.sync_copy(out_v, out_hbm.at[pl.ds(base_env * S * C, CE * S * C)])
        return carry

    lax.fori_loop(0, NCH, chunk_body, 0)


def kernel(tile_type, unit_counts_player_0, relic_map,
           normalized_reward_last_round, embed_table):
    tt = tile_type.reshape(E * S)
    uc = unit_counts_player_0.reshape(E * S)
    rm = relic_map.reshape(E * S)
    rw = jnp.repeat(normalized_reward_last_round.reshape(E), L)
    tab = embed_table.reshape(L)

    mesh = plsc.VectorSubcoreMesh(core_axis_name="c", subcore_axis_name="s")
    run = pl.kernel(
        _body,
        mesh=mesh,
        compiler_params=pltpu.CompilerParams(needs_layout_passes=False),
        out_type=jax.ShapeDtypeStruct((E * S * C,), jnp.float32),
        scratch_types=[
            pltpu.VMEM((CE * S,), jnp.int32),
            pltpu.VMEM((CE * S,), jnp.float32),
            pltpu.VMEM((CE * S,), jnp.int32),
            pltpu.VMEM((EPW * L,), jnp.float32),
            pltpu.VMEM((L,), jnp.float32),
            pltpu.VMEM((CE * S * C,), jnp.float32),
        ],
    )
    out = run(tt, uc, rm, rw, tab)
    return out.reshape(B, P, H, W, C)


# speedup vs baseline: 66.0303x; 14.4651x over previous
import functools

import jax
import jax.numpy as jnp
from jax import lax
from jax.experimental import pallas as pl
from jax.experimental.pallas import tpu as pltpu
from jax.experimental.pallas import tpu_sc as plsc

B, P, H, W = 4096, 2, 24, 24
C = 7
PH = P * H
W8 = W // 8
NB = B // 128
NW = 32
NT = 4
TW = 1024
UNITS = PH * W8 * (NB // NT)
UPW = UNITS // NW
NQ = NB // NT
L = 16


def _as_flat(x):
    x = jnp.transpose(x, (1, 2, 3, 0))
    x = x.reshape(P, H, W8, 8, NB, 128)
    x = jnp.transpose(x, (0, 1, 2, 4, 3, 5))
    return x.reshape(PH * W8 * NB * TW)


def _body(tt_hbm, uc_hbm, rm_hbm, rw_hbm, tab_hbm, out_hbm,
          tt_v, uc_v, rm_v, rw_v, tab_v, out_v):
    wid = lax.axis_index("s") * 2 + lax.axis_index("c")
    pltpu.sync_copy(tab_hbm, tab_v)
    pltpu.sync_copy(rw_hbm, rw_v)

    def unit_body(u, carry):
        q = u % NQ
        row = u // NQ
        p = u // (NQ * W8 * H)
        in_off = (row * NB + q * NT) * TW
        pltpu.sync_copy(tt_hbm.at[pl.ds(in_off, NT * TW)], tt_v)
        pltpu.sync_copy(uc_hbm.at[pl.ds(in_off, NT * TW)], uc_v)
        pltpu.sync_copy(rm_hbm.at[pl.ds(in_off, NT * TW)], rm_v)
        rw_base = q * (NT * 2 * 128) + p * 128
        for k in range(NT):
            for j in range(8):
                rv = rw_v[pl.ds(rw_base + k * 256 + j * L, L)]

                @plsc.parallel_loop(0, TW, 128, unroll=8)
                def sub_body(o128, k=k, j=j, rv=rv):
                    o = k * TW + o128 + j * L
                    sl = pl.ds(o, L)
                    tt = tt_v[sl]
                    uc = uc_v[sl]
                    rm = rm_v[sl]
                    out_v[pl.ds(0 * NT * TW + o, L)] = plsc.load_gather(tab_v, [tt])
                    out_v[pl.ds(1 * NT * TW + o, L)] = plsc.load_gather(tab_v, [tt + 4])
                    out_v[pl.ds(2 * NT * TW + o, L)] = plsc.load_gather(tab_v, [tt + 8])
                    out_v[pl.ds(3 * NT * TW + o, L)] = plsc.load_gather(tab_v, [tt + 12])
                    out_v[pl.ds(4 * NT * TW + o, L)] = uc
                    out_v[pl.ds(5 * NT * TW + o, L)] = rm.astype(jnp.float32)
                    out_v[pl.ds(6 * NT * TW + o, L)] = rv

        ph = row // W8
        w8 = row % W8
        for c in range(C):
            out_off = (((ph * C + c) * W8 + w8) * NB + q * NT) * TW
            pltpu.sync_copy(out_v.at[pl.ds(c * NT * TW, NT * TW)],
                            out_hbm.at[pl.ds(out_off, NT * TW)])
        return carry

    lax.fori_loop(wid * UPW, (wid + 1) * UPW, unit_body, 0)


def kernel(tile_type, unit_counts_player_0, relic_map,
           normalized_reward_last_round, embed_table):
    tt = _as_flat(tile_type)
    uc = _as_flat(unit_counts_player_0)
    rm = _as_flat(relic_map)
    rw = normalized_reward_last_round.reshape(NB, 128, P)
    rw = jnp.transpose(rw, (0, 2, 1)).reshape(NB * P * 128)
    tab = embed_table.T.reshape(L)

    mesh = plsc.VectorSubcoreMesh(core_axis_name="c", subcore_axis_name="s")
    run = pl.kernel(
        _body,
        mesh=mesh,
        compiler_params=pltpu.CompilerParams(needs_layout_passes=False),
        out_type=jax.ShapeDtypeStruct((PH * C * W8 * NB * TW,), jnp.float32),
        scratch_types=[
            pltpu.VMEM((NT * TW,), jnp.int32),
            pltpu.VMEM((NT * TW,), jnp.float32),
            pltpu.VMEM((NT * TW,), jnp.int32),
            pltpu.VMEM((NB * P * 128,), jnp.float32),
            pltpu.VMEM((L,), jnp.float32),
            pltpu.VMEM((C * NT * TW,), jnp.float32),
        ],
    )
    out = run(tt, uc, rm, rw, tab)
    out = out.reshape(P, H, C, W8, NB, 8, 128)
    out = out.transpose(4, 6, 0, 1, 3, 5, 2)
    return out.reshape(B, P, H, W, C)


# speedup vs baseline: 79.8161x; 1.2088x over previous
import functools

import jax
import jax.numpy as jnp
from jax import lax
from jax.experimental import pallas as pl
from jax.experimental.pallas import tpu as pltpu
from jax.experimental.pallas import tpu_sc as plsc

B, P, H, W = 4096, 2, 24, 24
C = 7
PH = P * H
W8 = W // 8
NB = B // 128
NW = 32
NT = 4
TW = 1024
SZ = NT * TW
UNITS = PH * W8 * (NB // NT)
UPW = UNITS // NW
NQ = NB // NT
L = 16


def _as_flat(x):
    x = jnp.transpose(x, (1, 2, 3, 0))
    x = x.reshape(P, H, W8, 8, NB, 128)
    x = jnp.transpose(x, (0, 1, 2, 4, 3, 5))
    return x.reshape(PH * W8 * NB * TW)


def _in_off(u):
    return (u // NQ * NB + u % NQ * NT) * TW


def _out_off(u, c):
    q = u % NQ
    row = u // NQ
    ph = row // W8
    w8 = row % W8
    return (((ph * C + c) * W8 + w8) * NB + q * NT) * TW


def _body(tt_hbm, uc_hbm, rm_hbm, rw_hbm, tab_hbm, out_hbm,
          tt_v, uc_v, rm_v, rw_v, tab_v, out_v, in_sems, out_sems):
    wid = lax.axis_index("s") * 2 + lax.axis_index("c")
    pltpu.sync_copy(tab_hbm, tab_v)
    pltpu.sync_copy(rw_hbm, rw_v)
    start = wid * UPW
    end = start + UPW

    def start_in(u, b):
        off = _in_off(u)
        pltpu.async_copy(tt_hbm.at[pl.ds(off, SZ)], tt_v.at[b], in_sems.at[b])
        pltpu.async_copy(uc_hbm.at[pl.ds(off, SZ)], uc_v.at[b], in_sems.at[b])
        pltpu.async_copy(rm_hbm.at[pl.ds(off, SZ)], rm_v.at[b], in_sems.at[b])

    def wait_in(u, b):
        off = _in_off(u)
        pltpu.make_async_copy(
            tt_hbm.at[pl.ds(off, SZ)], tt_v.at[b], in_sems.at[b]).wait()
        pltpu.make_async_copy(
            uc_hbm.at[pl.ds(off, SZ)], uc_v.at[b], in_sems.at[b]).wait()
        pltpu.make_async_copy(
            rm_hbm.at[pl.ds(off, SZ)], rm_v.at[b], in_sems.at[b]).wait()

    def start_out(u, b):
        for c in range(C):
            pltpu.async_copy(
                out_v.at[b, pl.ds(c * SZ, SZ)],
                out_hbm.at[pl.ds(_out_off(u, c), SZ)], out_sems.at[b])

    def wait_out(u, b):
        for c in range(C):
            pltpu.make_async_copy(
                out_v.at[b, pl.ds(c * SZ, SZ)],
                out_hbm.at[pl.ds(_out_off(u, c), SZ)], out_sems.at[b]).wait()

    def compute(u, b):
        q = u % NQ
        p = u // (NQ * W8 * H)
        rw_base = q * (NT * 2 * 128) + p * 128
        for k in range(NT):
            for j in range(8):
                rv = rw_v[pl.ds(rw_base + k * 256 + j * L, L)]

                @plsc.parallel_loop(0, TW, 128, unroll=8)
                def sub_body(o128, k=k, j=j, rv=rv, b=b):
                    o = k * TW + o128 + j * L
                    sl = pl.ds(o, L)
                    tt = tt_v[b, sl]
                    uc = uc_v[b, sl]
                    rm = rm_v[b, sl]
                    out_v[b, pl.ds(0 * SZ + o, L)] = plsc.load_gather(tab_v, [tt])
                    out_v[b, pl.ds(1 * SZ + o, L)] = plsc.load_gather(tab_v, [tt + 4])
                    out_v[b, pl.ds(2 * SZ + o, L)] = plsc.load_gather(tab_v, [tt + 8])
                    out_v[b, pl.ds(3 * SZ + o, L)] = plsc.load_gather(tab_v, [tt + 12])
                    out_v[b, pl.ds(4 * SZ + o, L)] = uc
                    out_v[b, pl.ds(5 * SZ + o, L)] = rm.astype(jnp.float32)
                    out_v[b, pl.ds(6 * SZ + o, L)] = rv

    start_in(start, 0)
    start_in(start + 1, 1)

    def pair_body(t, carry):
        g0 = start + 2 * t
        for b in range(2):
            u = g0 + b

            @pl.when(u >= start + 2)
            def _(u=u, b=b):
                wait_out(u - 2, b)

            wait_in(u, b)
            compute(u, b)
            start_out(u, b)

            @pl.when(u + 2 < end)
            def _(u=u, b=b):
                start_in(u + 2, b)

        return carry

    lax.fori_loop(0, UPW // 2, pair_body, 0)
    wait_out(end - 2, 0)
    wait_out(end - 1, 1)


def kernel(tile_type, unit_counts_player_0, relic_map,
           normalized_reward_last_round, embed_table):
    tt = _as_flat(tile_type)
    uc = _as_flat(unit_counts_player_0)
    rm = _as_flat(relic_map)
    rw = normalized_reward_last_round.reshape(NB, 128, P)
    rw = jnp.transpose(rw, (0, 2, 1)).reshape(NB * P * 128)
    tab = embed_table.T.reshape(L)

    mesh = plsc.VectorSubcoreMesh(core_axis_name="c", subcore_axis_name="s")
    run = pl.kernel(
        _body,
        mesh=mesh,
        compiler_params=pltpu.CompilerParams(needs_layout_passes=False),
        out_type=jax.ShapeDtypeStruct((PH * C * W8 * NB * TW,), jnp.float32),
        scratch_types=[
            pltpu.VMEM((2, SZ), jnp.int32),
            pltpu.VMEM((2, SZ), jnp.float32),
            pltpu.VMEM((2, SZ), jnp.int32),
            pltpu.VMEM((NB * P * 128,), jnp.float32),
            pltpu.VMEM((L,), jnp.float32),
            pltpu.VMEM((2, C * SZ), jnp.float32),
            pltpu.SemaphoreType.DMA((2,)),
            pltpu.SemaphoreType.DMA((2,)),
        ],
    )
    out = run(tt, uc, rm, rw, tab)
    out = out.reshape(P, H, C, W8, NB, 8, 128)
    out = out.transpose(4, 6, 0, 1, 3, 5, 2)
    return out.reshape(B, P, H, W, C)


# speedup vs baseline: 104.6676x; 1.3114x over previous
import functools

import jax
import jax.numpy as jnp
from jax import lax
from jax.experimental import pallas as pl
from jax.experimental.pallas import tpu as pltpu
from jax.experimental.pallas import tpu_sc as plsc

B, P, H, W = 4096, 2, 24, 24
C = 7
PH = P * H
W8 = W // 8
NB = B // 128
NW = 32
NT = 4
NR = NT * 8
ROWS = PH * W8 * NB * 8
UNITS = PH * W8 * (NB // NT)
UPW = UNITS // NW
NQ = NB // NT
L = 16


def _as_rows(x):
    x = jnp.transpose(x, (1, 2, 3, 0))
    x = x.reshape(P, H, W8, 8, NB, 128)
    x = jnp.transpose(x, (0, 1, 2, 4, 3, 5))
    return x.reshape(ROWS, 128)


def _in_row(u):
    return (u // NQ * NB + u % NQ * NT) * 8


def _body(tt_hbm, uc_hbm, rm_hbm, rw_hbm, tab_hbm, out_hbm,
          tt_v, uc_v, rm_v, rw_v, tab_v, out_v, in_sems, out_sems):
    wid = lax.axis_index("s") * 2 + lax.axis_index("c")
    pltpu.sync_copy(tab_hbm, tab_v)
    pltpu.sync_copy(rw_hbm, rw_v)
    start = wid * UPW
    end = start + UPW

    def out_slice(u):
        q = u % NQ
        row = u // NQ
        ph = row // W8
        w8 = row % W8
        return out_hbm.at[ph, :, pl.ds((w8 * NB + q * NT) * 8, NR), :]

    def start_in(u, b):
        r = _in_row(u)
        pltpu.async_copy(tt_hbm.at[pl.ds(r, NR), :], tt_v.at[b], in_sems.at[b])
        pltpu.async_copy(uc_hbm.at[pl.ds(r, NR), :], uc_v.at[b], in_sems.at[b])
        pltpu.async_copy(rm_hbm.at[pl.ds(r, NR), :], rm_v.at[b], in_sems.at[b])

    def wait_in(u, b):
        r = _in_row(u)
        pltpu.make_async_copy(
            tt_hbm.at[pl.ds(r, NR), :], tt_v.at[b], in_sems.at[b]).wait()
        pltpu.make_async_copy(
            uc_hbm.at[pl.ds(r, NR), :], uc_v.at[b], in_sems.at[b]).wait()
        pltpu.make_async_copy(
            rm_hbm.at[pl.ds(r, NR), :], rm_v.at[b], in_sems.at[b]).wait()

    def start_out(u, b):
        pltpu.async_copy(out_v.at[b], out_slice(u), out_sems.at[b])

    def wait_out(u, b):
        pltpu.make_async_copy(out_v.at[b], out_slice(u), out_sems.at[b]).wait()

    def compute(u, b):
        q = u % NQ
        p = u // (NQ * W8 * H)
        rw_base = q * (NT * 2 * 128) + p * 128
        for k in range(NT):
            for sub in range(8):
                row = k * 8 + sub

                @plsc.parallel_loop(0, 128, L, unroll=8)
                def lane_body(o16, k=k, row=row, b=b, rw_base=rw_base):
                    sl = pl.ds(o16, L)
                    tt = tt_v[b, row, sl]
                    uc = uc_v[b, row, sl]
                    rm = rm_v[b, row, sl]
                    rv = rw_v[pl.ds(rw_base + k * 256 + o16, L)]
                    out_v[b, 0, row, sl] = plsc.load_gather(tab_v, [tt])
                    out_v[b, 1, row, sl] = plsc.load_gather(tab_v, [tt + 4])
                    out_v[b, 2, row, sl] = plsc.load_gather(tab_v, [tt + 8])
                    out_v[b, 3, row, sl] = plsc.load_gather(tab_v, [tt + 12])
                    out_v[b, 4, row, sl] = uc
                    out_v[b, 5, row, sl] = rm.astype(jnp.float32)
                    out_v[b, 6, row, sl] = rv

    start_in(start, 0)
    start_in(start + 1, 1)

    def pair_body(t, carry):
        g0 = start + 2 * t
        for b in range(2):
            u = g0 + b

            @pl.when(u >= start + 2)
            def _(u=u, b=b):
                wait_out(u - 2, b)

            wait_in(u, b)
            compute(u, b)
            start_out(u, b)

            @pl.when(u + 2 < end)
            def _(u=u, b=b):
                start_in(u + 2, b)

        return carry

    lax.fori_loop(0, UPW // 2, pair_body, 0)
    wait_out(end - 2, 0)
    wait_out(end - 1, 1)


def kernel(tile_type, unit_counts_player_0, relic_map,
           normalized_reward_last_round, embed_table):
    tt = _as_rows(tile_type)
    uc = _as_rows(unit_counts_player_0)
    rm = _as_rows(relic_map)
    rw = normalized_reward_last_round.reshape(NB, 128, P)
    rw = jnp.transpose(rw, (0, 2, 1)).reshape(NB * P * 128)
    tab = embed_table.T.reshape(L)

    mesh = plsc.VectorSubcoreMesh(core_axis_name="c", subcore_axis_name="s")
    run = pl.kernel(
        _body,
        mesh=mesh,
        compiler_params=pltpu.CompilerParams(needs_layout_passes=False),
        out_type=jax.ShapeDtypeStruct((PH, C, W8 * NB * 8, 128), jnp.float32),
        scratch_types=[
            pltpu.VMEM((2, NR, 128), jnp.int32),
            pltpu.VMEM((2, NR, 128), jnp.float32),
            pltpu.VMEM((2, NR, 128), jnp.int32),
            pltpu.VMEM((NB * P * 128,), jnp.float32),
            pltpu.VMEM((L,), jnp.float32),
            pltpu.VMEM((2, C, NR, 128), jnp.float32),
            pltpu.SemaphoreType.DMA((2,)),
            pltpu.SemaphoreType.DMA((2,)),
        ],
    )
    out = run(tt, uc, rm, rw, tab)
    out = out.reshape(P, H, C, W8, NB, 8, 128)
    out = out.transpose(4, 6, 0, 1, 3, 5, 2)
    return out.reshape(B, P, H, W, C)


# speedup vs baseline: 118.2228x; 1.1295x over previous
import functools

import jax
import jax.numpy as jnp
from jax import lax
from jax.experimental import pallas as pl
from jax.experimental.pallas import tpu as pltpu
from jax.experimental.pallas import tpu_sc as plsc

B, P, H, W = 4096, 2, 24, 24
C = 7
PH = P * H
W8 = W // 8
NB = B // 128
NW = 32
NT = 4
NR = NT * 8
ROWS = PH * W8 * NB * 8
UNITS = PH * W8 * (NB // NT)
UPW = UNITS // NW
NQ = NB // NT
L = 16


def _as_rows(x):
    x = jnp.transpose(x, (1, 2, 3, 0))
    x = x.reshape(P, H, W8, 8, NB, 128)
    x = jnp.transpose(x, (0, 1, 2, 4, 3, 5))
    return x.reshape(ROWS, 128)


def _in_row(u):
    return (u // NQ * NB + u % NQ * NT) * 8


def _body(tt_hbm, uc_hbm, rm_hbm, rw_hbm, tab_hbm, out_hbm,
          tt_v, rm_v, rw_v, tab_v, out_v, in_sems, out_sems, uc_sems):
    wid = lax.axis_index("s") * 2 + lax.axis_index("c")
    pltpu.sync_copy(tab_hbm, tab_v)
    pltpu.sync_copy(rw_hbm, rw_v)
    start = wid * UPW
    end = start + UPW

    def out_slice(u):
        q = u % NQ
        row = u // NQ
        ph = row // W8
        w8 = row % W8
        return out_hbm.at[ph, :, pl.ds((w8 * NB + q * NT) * 8, NR), :]

    def start_in(u, b):
        r = _in_row(u)
        pltpu.async_copy(tt_hbm.at[pl.ds(r, NR), :], tt_v.at[b], in_sems.at[b])
        pltpu.async_copy(rm_hbm.at[pl.ds(r, NR), :], rm_v.at[b], in_sems.at[b])

    def wait_in(u, b):
        r = _in_row(u)
        pltpu.make_async_copy(
            tt_hbm.at[pl.ds(r, NR), :], tt_v.at[b], in_sems.at[b]).wait()
        pltpu.make_async_copy(
            rm_hbm.at[pl.ds(r, NR), :], rm_v.at[b], in_sems.at[b]).wait()

    def start_uc(u, b):
        pltpu.async_copy(uc_hbm.at[pl.ds(_in_row(u), NR), :], out_v.at[b, 4],
                         uc_sems.at[b])

    def wait_uc(u, b):
        pltpu.make_async_copy(
            uc_hbm.at[pl.ds(_in_row(u), NR), :], out_v.at[b, 4],
            uc_sems.at[b]).wait()

    def start_out(u, b):
        pltpu.async_copy(out_v.at[b], out_slice(u), out_sems.at[b])

    def wait_out(u, b):
        pltpu.make_async_copy(out_v.at[b], out_slice(u), out_sems.at[b]).wait()

    def compute(u, b):
        q = u % NQ
        p = u // (NQ * W8 * H)
        rw_base = q * (NT * 2 * 128) + p * 128
        for k in range(NT):
            for sub in range(8):
                row = k * 8 + sub

                @plsc.parallel_loop(0, 128, L, unroll=8)
                def lane_body(o16, k=k, row=row, b=b, rw_base=rw_base):
                    sl = pl.ds(o16, L)
                    tt = tt_v[b, row, sl]
                    rm = rm_v[b, row, sl]
                    rv = rw_v[pl.ds(rw_base + k * 256 + o16, L)]
                    out_v[b, 0, row, sl] = plsc.load_gather(tab_v, [tt])
                    out_v[b, 1, row, sl] = plsc.load_gather(tab_v, [tt + 4])
                    out_v[b, 2, row, sl] = plsc.load_gather(tab_v, [tt + 8])
                    out_v[b, 3, row, sl] = plsc.load_gather(tab_v, [tt + 12])
                    out_v[b, 5, row, sl] = rm.astype(jnp.float32)
                    out_v[b, 6, row, sl] = rv

    start_in(start, 0)
    start_in(start + 1, 1)

    def pair_body(t, carry):
        g0 = start + 2 * t
        for b in range(2):
            u = g0 + b

            @pl.when(u >= start + 2)
            def _(u=u, b=b):
                wait_out(u - 2, b)

            start_uc(u, b)
            wait_in(u, b)
            compute(u, b)
            wait_uc(u, b)
            start_out(u, b)

            @pl.when(u + 2 < end)
            def _(u=u, b=b):
                start_in(u + 2, b)

        return carry

    lax.fori_loop(0, UPW // 2, pair_body, 0)
    wait_out(end - 2, 0)
    wait_out(end - 1, 1)


def kernel(tile_type, unit_counts_player_0, relic_map,
           normalized_reward_last_round, embed_table):
    tt = _as_rows(tile_type)
    uc = _as_rows(unit_counts_player_0)
    rm = _as_rows(relic_map)
    rw = normalized_reward_last_round.reshape(NB, 128, P)
    rw = jnp.transpose(rw, (0, 2, 1)).reshape(NB * P * 128)
    tab = embed_table.T.reshape(L)

    mesh = plsc.VectorSubcoreMesh(core_axis_name="c", subcore_axis_name="s")
    run = pl.kernel(
        _body,
        mesh=mesh,
        compiler_params=pltpu.CompilerParams(needs_layout_passes=False),
        out_type=jax.ShapeDtypeStruct((PH, C, W8 * NB * 8, 128), jnp.float32),
        scratch_types=[
            pltpu.VMEM((2, NR, 128), jnp.int32),
            pltpu.VMEM((2, NR, 128), jnp.int32),
            pltpu.VMEM((NB * P * 128,), jnp.float32),
            pltpu.VMEM((L,), jnp.float32),
            pltpu.VMEM((2, C, NR, 128), jnp.float32),
            pltpu.SemaphoreType.DMA((2,)),
            pltpu.SemaphoreType.DMA((2,)),
            pltpu.SemaphoreType.DMA((2,)),
        ],
    )
    out = run(tt, uc, rm, rw, tab)
    out = out.reshape(P, H, C, W8, NB, 8, 128)
    out = out.transpose(4, 6, 0, 1, 3, 5, 2)
    return out.reshape(B, P, H, W, C)


# speedup vs baseline: 146.6906x; 1.2408x over previous
import functools

import jax
import jax.numpy as jnp
from jax import lax
from jax.experimental import pallas as pl
from jax.experimental.pallas import tpu as pltpu
from jax.experimental.pallas import tpu_sc as plsc

B, P, H, W = 4096, 2, 24, 24
C = 7
PH = P * H
W8 = W // 8
NB = B // 128
NW = 32
NT = 4
NR = NT * 8
ROWS = PH * W8 * NB * 8
UNITS = PH * W8 * (NB // NT)
UPW = UNITS // NW
NQ = NB // NT
L = 16


def _as_rows(x):
    x = jnp.transpose(x, (1, 2, 3, 0))
    x = x.reshape(P, H, W8, 8, NB, 128)
    x = jnp.transpose(x, (0, 1, 2, 4, 3, 5))
    return x.reshape(ROWS, 128)


def _in_row(u):
    return (u // NQ * NB + u % NQ * NT) * 8


def _body(tt_hbm, uc_hbm, rm_hbm, rw_hbm, tab_hbm, out_hbm,
          tt_v, rm_v, rw_v, tab_v, out_v, in_sems, out_sems, uc_sems):
    wid = lax.axis_index("s") * 2 + lax.axis_index("c")
    pltpu.sync_copy(tab_hbm, tab_v)
    pltpu.sync_copy(rw_hbm, rw_v)
    start = wid * UPW
    end = start + UPW

    def out_slice(u):
        q = u % NQ
        row = u // NQ
        ph = row // W8
        w8 = row % W8
        return out_hbm.at[ph, :, pl.ds((w8 * NB + q * NT) * 8, NR), :]

    def start_in(u, b):
        r = _in_row(u)
        pltpu.async_copy(tt_hbm.at[pl.ds(r, NR), :], tt_v.at[b], in_sems.at[b])
        pltpu.async_copy(rm_hbm.at[pl.ds(r, NR), :], rm_v.at[b], in_sems.at[b])

    def wait_in(u, b):
        r = _in_row(u)
        pltpu.make_async_copy(
            tt_hbm.at[pl.ds(r, NR), :], tt_v.at[b], in_sems.at[b]).wait()
        pltpu.make_async_copy(
            rm_hbm.at[pl.ds(r, NR), :], rm_v.at[b], in_sems.at[b]).wait()

    def start_uc(u, b):
        pltpu.async_copy(uc_hbm.at[pl.ds(_in_row(u), NR), :], out_v.at[b, 4],
                         uc_sems.at[b])

    def wait_uc(u, b):
        pltpu.make_async_copy(
            uc_hbm.at[pl.ds(_in_row(u), NR), :], out_v.at[b, 4],
            uc_sems.at[b]).wait()

    def start_out(u, b):
        pltpu.async_copy(out_v.at[b], out_slice(u), out_sems.at[b])

    def wait_out(u, b):
        pltpu.make_async_copy(out_v.at[b], out_slice(u), out_sems.at[b]).wait()

    tvec = tab_v[pl.ds(0, L)]
    tb = [[jnp.full((L,), tvec[c * 4 + r], jnp.float32) for r in range(4)]
          for c in range(4)]

    def compute(u, b):
        q = u % NQ
        p = u // (NQ * W8 * H)
        rw_base = q * (NT * 2 * 128) + p * 128
        for k in range(NT):
            for sub in range(8):
                row = k * 8 + sub

                @plsc.parallel_loop(0, 128, L, unroll=8)
                def lane_body(o16, k=k, row=row, b=b, rw_base=rw_base):
                    sl = pl.ds(o16, L)
                    tt = tt_v[b, row, sl]
                    rm = rm_v[b, row, sl]
                    rv = rw_v[pl.ds(rw_base + k * 256 + o16, L)]
                    m1 = tt >= 1
                    m2 = tt >= 2
                    m3 = tt >= 3
                    for c in range(4):
                        e = jnp.where(m1, tb[c][1], tb[c][0])
                        e = jnp.where(m2, tb[c][2], e)
                        e = jnp.where(m3, tb[c][3], e)
                        out_v[b, c, row, sl] = e
                    out_v[b, 5, row, sl] = rm.astype(jnp.float32)
                    out_v[b, 6, row, sl] = rv

    start_in(start, 0)
    start_in(start + 1, 1)

    def pair_body(t, carry):
        g0 = start + 2 * t
        for b in range(2):
            u = g0 + b

            @pl.when(u >= start + 2)
            def _(u=u, b=b):
                wait_out(u - 2, b)

            start_uc(u, b)
            wait_in(u, b)
            compute(u, b)
            wait_uc(u, b)
            start_out(u, b)

            @pl.when(u + 2 < end)
            def _(u=u, b=b):
                start_in(u + 2, b)

        return carry

    lax.fori_loop(0, UPW // 2, pair_body, 0)
    wait_out(end - 2, 0)
    wait_out(end - 1, 1)


def kernel(tile_type, unit_counts_player_0, relic_map,
           normalized_reward_last_round, embed_table):
    tt = _as_rows(tile_type)
    uc = _as_rows(unit_counts_player_0)
    rm = _as_rows(relic_map)
    rw = normalized_reward_last_round.reshape(NB, 128, P)
    rw = jnp.transpose(rw, (0, 2, 1)).reshape(NB * P * 128)
    tab = embed_table.T.reshape(L)

    mesh = plsc.VectorSubcoreMesh(core_axis_name="c", subcore_axis_name="s")
    run = pl.kernel(
        _body,
        mesh=mesh,
        compiler_params=pltpu.CompilerParams(needs_layout_passes=False),
        out_type=jax.ShapeDtypeStruct((PH, C, W8 * NB * 8, 128), jnp.float32),
        scratch_types=[
            pltpu.VMEM((2, NR, 128), jnp.int32),
            pltpu.VMEM((2, NR, 128), jnp.int32),
            pltpu.VMEM((NB * P * 128,), jnp.float32),
            pltpu.VMEM((L,), jnp.float32),
            pltpu.VMEM((2, C, NR, 128), jnp.float32),
            pltpu.SemaphoreType.DMA((2,)),
            pltpu.SemaphoreType.DMA((2,)),
            pltpu.SemaphoreType.DMA((2,)),
        ],
    )
    out = run(tt, uc, rm, rw, tab)
    out = out.reshape(P, H, C, W8, NB, 8, 128)
    out = out.transpose(4, 6, 0, 1, 3, 5, 2)
    return out.reshape(B, P, H, W, C)


# speedup vs baseline: 153.5236x; 1.0466x over previous
import functools

import jax
import jax.numpy as jnp
from jax import lax
from jax.experimental import pallas as pl
from jax.experimental.pallas import tpu as pltpu
from jax.experimental.pallas import tpu_sc as plsc

B, P, H, W = 4096, 2, 24, 24
C = 7
PH = P * H
W8 = W // 8
NB = B // 128
NW = 32
NT = 4
NR = NT * 8
ROWS = PH * W8 * NB * 8
UNITS = PH * W8 * (NB // NT)
UPW = UNITS // NW
NQ = NB // NT
L = 16


def _as_rows(x):
    x = jnp.transpose(x, (1, 2, 3, 0))
    x = x.reshape(P, H, W8, 8, NB, 128)
    x = jnp.transpose(x, (0, 1, 2, 4, 3, 5))
    return x.reshape(ROWS, 128)


def _in_row(u):
    return (u // NQ * NB + u % NQ * NT) * 8


def _body(tt_hbm, uc_hbm, rm_hbm, rw_hbm, tab_hbm, out_hbm,
          tt_v, rm_v, rw_v, tab_v, out_v, rv_slab, in_sems, out_sems, uc_sems):
    wid = lax.axis_index("s") * 2 + lax.axis_index("c")
    pltpu.sync_copy(tab_hbm, tab_v)
    pltpu.sync_copy(rw_hbm, rw_v)
    start = wid * UPW
    end = start + UPW
    p = wid // 16

    for q in range(NQ):
        for k in range(NT):
            for j in range(8):
                val = rw_v[pl.ds(q * NT * P * 128 + k * P * 128 + p * 128
                                 + j * L, L)]
                for sub in range(8):
                    rv_slab[q, k * 8 + sub, pl.ds(j * L, L)] = val

    def out_slice6(u):
        q = u % NQ
        row = u // NQ
        ph = row // W8
        w8 = row % W8
        rr = (w8 * NB + q * NT) * 8
        return (out_hbm.at[ph, pl.ds(0, 6), pl.ds(rr, NR), :],
                out_hbm.at[ph, 6, pl.ds(rr, NR), :])

    def start_in(u, b):
        r = _in_row(u)
        pltpu.async_copy(tt_hbm.at[pl.ds(r, NR), :], tt_v.at[b], in_sems.at[b])
        pltpu.async_copy(rm_hbm.at[pl.ds(r, NR), :], rm_v.at[b], in_sems.at[b])

    def wait_in(u, b):
        r = _in_row(u)
        pltpu.make_async_copy(
            tt_hbm.at[pl.ds(r, NR), :], tt_v.at[b], in_sems.at[b]).wait()
        pltpu.make_async_copy(
            rm_hbm.at[pl.ds(r, NR), :], rm_v.at[b], in_sems.at[b]).wait()

    def start_uc(u, b):
        pltpu.async_copy(uc_hbm.at[pl.ds(_in_row(u), NR), :], out_v.at[b, 4],
                         uc_sems.at[b])

    def wait_uc(u, b):
        pltpu.make_async_copy(
            uc_hbm.at[pl.ds(_in_row(u), NR), :], out_v.at[b, 4],
            uc_sems.at[b]).wait()

    def start_out(u, b):
        d6, dr = out_slice6(u)
        pltpu.async_copy(out_v.at[b], d6, out_sems.at[b])
        pltpu.async_copy(rv_slab.at[u % NQ], dr, out_sems.at[b])

    def wait_out(u, b):
        d6, dr = out_slice6(u)
        pltpu.make_async_copy(out_v.at[b], d6, out_sems.at[b]).wait()
        pltpu.make_async_copy(rv_slab.at[u % NQ], dr, out_sems.at[b]).wait()

    tvec = tab_v[pl.ds(0, L)]
    tb = [[jnp.full((L,), tvec[c * 4 + r], jnp.float32) for r in range(4)]
          for c in range(4)]

    def compute(u, b):
        for row in range(NR):

            @plsc.parallel_loop(0, 128, L, unroll=8)
            def lane_body(o16, row=row, b=b):
                sl = pl.ds(o16, L)
                tt = tt_v[b, row, sl]
                rm = rm_v[b, row, sl]
                m1 = tt >= 1
                m2 = tt >= 2
                m3 = tt >= 3
                for c in range(4):
                    e = jnp.where(m1, tb[c][1], tb[c][0])
                    e = jnp.where(m2, tb[c][2], e)
                    e = jnp.where(m3, tb[c][3], e)
                    out_v[b, c, row, sl] = e
                out_v[b, 5, row, sl] = rm.astype(jnp.float32)

    start_in(start, 0)
    start_in(start + 1, 1)

    def pair_body(t, carry):
        g0 = start + 2 * t
        for b in range(2):
            u = g0 + b

            @pl.when(u >= start + 2)
            def _(u=u, b=b):
                wait_out(u - 2, b)

            start_uc(u, b)
            wait_in(u, b)
            compute(u, b)
            wait_uc(u, b)
            start_out(u, b)

            @pl.when(u + 2 < end)
            def _(u=u, b=b):
                start_in(u + 2, b)

        return carry

    lax.fori_loop(0, UPW // 2, pair_body, 0)
    wait_out(end - 2, 0)
    wait_out(end - 1, 1)


def kernel(tile_type, unit_counts_player_0, relic_map,
           normalized_reward_last_round, embed_table):
    tt = _as_rows(tile_type)
    uc = _as_rows(unit_counts_player_0)
    rm = _as_rows(relic_map)
    rw = normalized_reward_last_round.reshape(NB, 128, P)
    rw = jnp.transpose(rw, (0, 2, 1)).reshape(NB * P * 128)
    tab = embed_table.T.reshape(L)

    mesh = plsc.VectorSubcoreMesh(core_axis_name="c", subcore_axis_name="s")
    run = pl.kernel(
        _body,
        mesh=mesh,
        compiler_params=pltpu.CompilerParams(needs_layout_passes=False),
        out_type=jax.ShapeDtypeStruct((PH, C, W8 * NB * 8, 128), jnp.float32),
        scratch_types=[
            pltpu.VMEM((2, NR, 128), jnp.int32),
            pltpu.VMEM((2, NR, 128), jnp.int32),
            pltpu.VMEM((NB * P * 128,), jnp.float32),
            pltpu.VMEM((L,), jnp.float32),
            pltpu.VMEM((2, 6, NR, 128), jnp.float32),
            pltpu.VMEM((NQ, NR, 128), jnp.float32),
            pltpu.SemaphoreType.DMA((2,)),
            pltpu.SemaphoreType.DMA((2,)),
            pltpu.SemaphoreType.DMA((2,)),
        ],
    )
    out = run(tt, uc, rm, rw, tab)
    out = out.reshape(P, H, C, W8, NB, 8, 128)
    out = out.transpose(4, 6, 0, 1, 3, 5, 2)
    return out.reshape(B, P, H, W, C)


# speedup vs baseline: 154.8314x; 1.0085x over previous
import jax
import jax.numpy as jnp
from jax import lax
from jax.experimental import pallas as pl
from jax.experimental.pallas import tpu as pltpu
from jax.experimental.pallas import tpu_sc as plsc

B, P, H, W = 4096, 2, 24, 24
C = 7
PH = P * H
W8 = W // 8
NB = B // 128
NW = 32
NT = 4
NR = NT * 8
ROWS = PH * W8 * NB * 8
UNITS = PH * W8 * (NB // NT)
UPW = UNITS // NW
NQ = NB // NT
L = 16


def _as_rows(x):
    x = jnp.transpose(x, (1, 2, 3, 0))
    x = x.reshape(P, H, W8, 8, NB, 128)
    x = jnp.transpose(x, (0, 1, 2, 4, 3, 5))
    return x.reshape(ROWS, 128)


def _in_row(u):
    return (u // NQ * NB + u % NQ * NT) * 8


def _body(tt_hbm, uc_hbm, rm_hbm, rw_hbm, tab_hbm, out_hbm,
          tt_v, rm_v, rw_v, tab_v, out_v, rv_slab, in_sems, out_sems, uc_sems):
    wid = lax.axis_index("s") * 2 + lax.axis_index("c")
    pltpu.sync_copy(tab_hbm, tab_v)
    pltpu.sync_copy(rw_hbm, rw_v)
    start = wid * UPW
    end = start + UPW
    p = wid // 16

    def _prefetch(u, b):
        r = _in_row(u)
        pltpu.async_copy(tt_hbm.at[pl.ds(r, NR), :], tt_v.at[b], in_sems.at[b])
        pltpu.async_copy(rm_hbm.at[pl.ds(r, NR), :], rm_v.at[b], in_sems.at[b])

    _prefetch(start, 0)
    _prefetch(start + 1, 1)

    for q in range(NQ):
        for k in range(NT):
            for j in range(8):
                val = rw_v[pl.ds(q * NT * P * 128 + k * P * 128 + p * 128
                                 + j * L, L)]
                for sub in range(8):
                    rv_slab[q, k * 8 + sub, pl.ds(j * L, L)] = val

    def out_slice6(u):
        q = u % NQ
        row = u // NQ
        ph = row // W8
        w8 = row % W8
        rr = (w8 * NB + q * NT) * 8
        return (out_hbm.at[ph, pl.ds(0, 6), pl.ds(rr, NR), :],
                out_hbm.at[ph, 6, pl.ds(rr, NR), :])

    def start_in(u, b):
        r = _in_row(u)
        pltpu.async_copy(tt_hbm.at[pl.ds(r, NR), :], tt_v.at[b], in_sems.at[b])
        pltpu.async_copy(rm_hbm.at[pl.ds(r, NR), :], rm_v.at[b], in_sems.at[b])

    def wait_in(u, b):
        r = _in_row(u)
        pltpu.make_async_copy(
            tt_hbm.at[pl.ds(r, NR), :], tt_v.at[b], in_sems.at[b]).wait()
        pltpu.make_async_copy(
            rm_hbm.at[pl.ds(r, NR), :], rm_v.at[b], in_sems.at[b]).wait()

    def start_uc(u, b):
        pltpu.async_copy(uc_hbm.at[pl.ds(_in_row(u), NR), :], out_v.at[b, 4],
                         uc_sems.at[b])

    def wait_uc(u, b):
        pltpu.make_async_copy(
            uc_hbm.at[pl.ds(_in_row(u), NR), :], out_v.at[b, 4],
            uc_sems.at[b]).wait()

    def start_out(u, b):
        d6, dr = out_slice6(u)
        pltpu.async_copy(out_v.at[b], d6, out_sems.at[b])
        pltpu.async_copy(rv_slab.at[u % NQ], dr, out_sems.at[b])

    def wait_out(u, b):
        d6, dr = out_slice6(u)
        pltpu.make_async_copy(out_v.at[b], d6, out_sems.at[b]).wait()
        pltpu.make_async_copy(rv_slab.at[u % NQ], dr, out_sems.at[b]).wait()

    tvec = tab_v[pl.ds(0, L)]
    tb = [[jnp.full((L,), tvec[c * 4 + r], jnp.float32) for r in range(4)]
          for c in range(4)]

    def compute(u, b):
        for row in range(NR):

            @plsc.parallel_loop(0, 128, L, unroll=8)
            def lane_body(o16, row=row, b=b):
                sl = pl.ds(o16, L)
                tt = tt_v[b, row, sl]
                rm = rm_v[b, row, sl]
                m1 = tt >= 1
                m2 = tt >= 2
                m3 = tt >= 3
                for c in range(4):
                    e = jnp.where(m1, tb[c][1], tb[c][0])
                    e = jnp.where(m2, tb[c][2], e)
                    e = jnp.where(m3, tb[c][3], e)
                    out_v[b, c, row, sl] = e
                out_v[b, 5, row, sl] = rm.astype(jnp.float32)

    def pair_body(t, carry):
        g0 = start + 2 * t
        for b in range(2):
            u = g0 + b

            @pl.when(u >= start + 2)
            def _(u=u, b=b):
                wait_out(u - 2, b)

            start_uc(u, b)
            wait_in(u, b)
            compute(u, b)
            wait_uc(u, b)
            start_out(u, b)

            @pl.when(u + 2 < end)
            def _(u=u, b=b):
                start_in(u + 2, b)

        return carry

    lax.fori_loop(0, UPW // 2, pair_body, 0)
    wait_out(end - 2, 0)
    wait_out(end - 1, 1)


def kernel(tile_type, unit_counts_player_0, relic_map,
           normalized_reward_last_round, embed_table):
    tt = _as_rows(tile_type)
    uc = _as_rows(unit_counts_player_0)
    rm = _as_rows(relic_map)
    rw = normalized_reward_last_round.reshape(NB, 128, P)
    rw = jnp.transpose(rw, (0, 2, 1)).reshape(NB * P * 128)
    tab = embed_table.T.reshape(L)

    mesh = plsc.VectorSubcoreMesh(core_axis_name="c", subcore_axis_name="s")
    run = pl.kernel(
        _body,
        mesh=mesh,
        compiler_params=pltpu.CompilerParams(needs_layout_passes=False),
        out_type=jax.ShapeDtypeStruct((PH, C, W8 * NB * 8, 128), jnp.float32),
        scratch_types=[
            pltpu.VMEM((2, NR, 128), jnp.int32),
            pltpu.VMEM((2, NR, 128), jnp.int32),
            pltpu.VMEM((NB * P * 128,), jnp.float32),
            pltpu.VMEM((L,), jnp.float32),
            pltpu.VMEM((2, 6, NR, 128), jnp.float32),
            pltpu.VMEM((NQ, NR, 128), jnp.float32),
            pltpu.SemaphoreType.DMA((2,)),
            pltpu.SemaphoreType.DMA((2,)),
            pltpu.SemaphoreType.DMA((2,)),
        ],
    )
    out = run(tt, uc, rm, rw, tab)
    out = out.reshape(P, H, C, W8, NB, 8, 128)
    out = out.transpose(4, 6, 0, 1, 3, 5, 2)
    return out.reshape(B, P, H, W, C)
